# baseline (speedup 1.0000x reference)
import functools

import jax
import jax.numpy as jnp
from jax import lax
from jax.experimental import pallas as pl
from jax.experimental.pallas import tpu as pltpu
from jax.experimental.pallas import tpu_sc as plsc

_NUM_GROUP = 128
_GROUP_SIZE = 32
_EXPAND = 1.5
_T = 4


def _fps_body(xt_ref, cent_ref):
    _, b, n = xt_ref.shape
    ncen = cent_ref.shape[2]
    xs = xt_ref[0]
    ys = xt_ref[1]
    zs = xt_ref[2]
    lane = lax.broadcasted_iota(jnp.int32, (b, n), 1)
    cidx = lax.broadcasted_iota(jnp.int32, (b, ncen), 1)

    def body(i, carry):
        dists, last, cx, cy, cz = carry
        hot = lane == last
        px = jnp.sum(jnp.where(hot, xs, 0.0), axis=1, keepdims=True)
        py = jnp.sum(jnp.where(hot, ys, 0.0), axis=1, keepdims=True)
        pz = jnp.sum(jnp.where(hot, zs, 0.0), axis=1, keepdims=True)
        chere = cidx == (i - 1)
        cx = jnp.where(chere, px, cx)
        cy = jnp.where(chere, py, cy)
        cz = jnp.where(chere, pz, cz)
        dx = xs - px
        dy = ys - py
        dz = zs - pz
        d = dx * dx + dy * dy + dz * dz
        dists = jnp.minimum(dists, d)
        m = jnp.max(dists, axis=1, keepdims=True)
        nxt = jnp.min(jnp.where(dists == m, lane, n), axis=1, keepdims=True)
        return dists, nxt, cx, cy, cz

    dists0 = jnp.full((b, n), 1e10, jnp.float32)
    last0 = jnp.zeros((b, 1), jnp.int32)
    cz0 = jnp.zeros((b, ncen), jnp.float32)
    _, _, cx, cy, cz = lax.fori_loop(
        1, ncen + 1, body, (dists0, last0, cz0, cz0, cz0))
    cent_ref[0] = cx
    cent_ref[1] = cy
    cent_ref[2] = cz


def _knn_body(xt_ref, q_ref, idx_ref):
    n = xt_ref.shape[2]
    gs = idx_ref.shape[2]
    xt = xt_ref[0]
    q = q_ref[0]
    r0 = xt[0:1, :]
    r1 = xt[1:2, :]
    r2 = xt[2:3, :]
    q0 = q[:, 0:1]
    q1 = q[:, 1:2]
    q2 = q[:, 2:3]
    xtb = xt.astype(jnp.bfloat16).astype(jnp.float32)
    qb = q.astype(jnp.bfloat16).astype(jnp.float32)
    b0 = xtb[0:1, :]
    b1 = xtb[1:2, :]
    b2 = xtb[2:3, :]
    p0 = qb[:, 0:1]
    p1 = qb[:, 1:2]
    p2 = qb[:, 2:3]
    qr = p0 * b0 + p1 * b1 + p2 * b2
    qq = q0 * q0 + q1 * q1 + q2 * q2
    rr = r0 * r0 + r1 * r1 + r2 * r2
    d = (qq - 2.0 * qr) + rr
    lane = lax.broadcasted_iota(jnp.int32, d.shape, 1)
    nq = d.shape[0]
    nchunk = n // 128
    mm = jnp.min(d.reshape(nq, nchunk, 128), axis=2)
    for j in range(gs):
        m = jnp.min(mm, axis=1, keepdims=True)
        sel = jnp.min(jnp.where(d == m, lane, n), axis=1, keepdims=True)
        idx_ref[0, :, j:j + 1] = sel
        d = jnp.where(lane == sel, jnp.inf, d)
        mm = jnp.min(d.reshape(nq, nchunk, 128), axis=2)


def _make_sc_gather(ncloud, n, nidx):
    mesh = plsc.VectorSubcoreMesh(core_axis_name="c", subcore_axis_name="s")

    @functools.partial(
        pl.kernel,
        out_type=jax.ShapeDtypeStruct((ncloud, 3 * nidx), jnp.float32),
        mesh=mesh,
        scratch_types=[
            pltpu.VMEM((3 * n,), jnp.float32),
            pltpu.VMEM((nidx,), jnp.int32),
            pltpu.VMEM((3 * nidx,), jnp.float32),
            pltpu.VMEM((3 * nidx,), jnp.float32),
        ],
        compiler_params=pltpu.CompilerParams(needs_layout_passes=False),
    )
    def sc_gather(x_hbm, idx_hbm, cexp_hbm, out_hbm, xv, iv, cv, ov):
        w = lax.axis_index("s") * 2 + lax.axis_index("c")
        pltpu.sync_copy(x_hbm.at[w], xv)
        pltpu.sync_copy(idx_hbm.at[w], iv)
        pltpu.sync_copy(cexp_hbm.at[w], cv)

        def step(i, carry):
            off = pl.multiple_of(i * 16, 16)
            ivec = iv[pl.ds(off, 16)]
            for c3 in range(3):
                g = plsc.load_gather(xv, [ivec + jnp.int32(c3 * n)])
                off2 = pl.multiple_of(c3 * nidx + i * 16, 16)
                ov[pl.ds(off2, 16)] = g - cv[pl.ds(off2, 16)]
            return carry

        lax.fori_loop(0, nidx // 16, step, 0)
        pltpu.sync_copy(ov, out_hbm.at[w])

    return sc_gather


def _build_queries(centers, t):
    step_f = int((_EXPAND - 1.0) * _NUM_GROUP / t * 2)
    step_b = int((_EXPAND - 1.0) * _NUM_GROUP)
    parts = []
    for i in range(t):
        a = centers[:, i * step_f:i * step_f + (_NUM_GROUP - step_b)]
        b = centers[:, (i - 1) * step_b + _NUM_GROUP + (t - 1) * step_f:
                    i * step_b + _NUM_GROUP + (t - 1) * step_f]
        parts.append(jnp.concatenate((a, b), axis=1))
    return jnp.stack(parts, axis=0)


def kernel(xyz):
    t, b, n, _ = xyz.shape
    gs = _GROUP_SIZE
    step_f = int((_EXPAND - 1.0) * _NUM_GROUP / t * 2)
    step_b = int((_EXPAND - 1.0) * _NUM_GROUP)
    ncen = _NUM_GROUP + (step_f + step_b) * (t - 1)
    nc = t * b

    x = xyz.reshape(nc, n, 3)
    xt = jnp.transpose(x, (0, 2, 1))

    x8t = jnp.transpose(x[:b], (2, 0, 1))
    cent = pl.pallas_call(
        _fps_body,
        out_shape=jax.ShapeDtypeStruct((3, b, ncen), jnp.float32),
    )(x8t)
    centers = jnp.transpose(cent, (1, 2, 0))

    queries = _build_queries(centers, t).reshape(nc, -1, 3)
    m = queries.shape[1]

    idx = pl.pallas_call(
        _knn_body,
        grid=(nc,),
        in_specs=[
            pl.BlockSpec((1, 3, n), lambda g: (g, 0, 0)),
            pl.BlockSpec((1, m, 3), lambda g: (g, 0, 0)),
        ],
        out_specs=pl.BlockSpec((1, m, gs), lambda g: (g, 0, 0)),
        out_shape=jax.ShapeDtypeStruct((nc, m, gs), jnp.int32),
    )(xt, queries)

    xflat = xt.reshape(nc, 3 * n)
    idxflat = idx.reshape(nc, m * gs)
    qt = jnp.transpose(queries, (0, 2, 1))
    cexp = jnp.broadcast_to(
        qt[:, :, :, None], (nc, 3, m, gs)).reshape(nc, 3 * m * gs)
    nb = _make_sc_gather(nc, n, m * gs)(xflat, idxflat, cexp)

    neighborhood = nb.reshape(nc, 3, m, gs).transpose(0, 2, 3, 1)
    neighborhood = neighborhood.reshape(t, b, m, gs, 3)
    center_out = queries.reshape(t, b, m, 3)
    return (neighborhood, center_out)


# speedup vs baseline: 1.5290x; 1.5290x over previous
import functools

import jax
import jax.numpy as jnp
from jax import lax
from jax.experimental import pallas as pl
from jax.experimental.pallas import tpu as pltpu
from jax.experimental.pallas import tpu_sc as plsc

_NUM_GROUP = 128
_GROUP_SIZE = 32
_EXPAND = 1.5
_T = 4


def _fps_body(xt_ref, cent_ref):
    _, b, n = xt_ref.shape
    ncen = cent_ref.shape[2]
    xs = xt_ref[0]
    ys = xt_ref[1]
    zs = xt_ref[2]
    lane = lax.broadcasted_iota(jnp.int32, (b, n), 1)
    cidx = lax.broadcasted_iota(jnp.int32, (b, ncen), 1)

    def body(i, carry):
        dists, last, cx, cy, cz = carry
        hot = lane == last
        px = jnp.sum(jnp.where(hot, xs, 0.0), axis=1, keepdims=True)
        py = jnp.sum(jnp.where(hot, ys, 0.0), axis=1, keepdims=True)
        pz = jnp.sum(jnp.where(hot, zs, 0.0), axis=1, keepdims=True)
        chere = cidx == (i - 1)
        cx = jnp.where(chere, px, cx)
        cy = jnp.where(chere, py, cy)
        cz = jnp.where(chere, pz, cz)
        dx = xs - px
        dy = ys - py
        dz = zs - pz
        d = dx * dx + dy * dy + dz * dz
        dists = jnp.minimum(dists, d)
        m = jnp.max(dists, axis=1, keepdims=True)
        nxt = jnp.min(jnp.where(dists == m, lane, n), axis=1, keepdims=True)
        return dists, nxt, cx, cy, cz

    dists0 = jnp.full((b, n), 1e10, jnp.float32)
    last0 = jnp.zeros((b, 1), jnp.int32)
    cz0 = jnp.zeros((b, ncen), jnp.float32)
    _, _, cx, cy, cz = lax.fori_loop(
        1, ncen + 1, body, (dists0, last0, cz0, cz0, cz0))
    cent_ref[0] = cx
    cent_ref[1] = cy
    cent_ref[2] = cz


def _knn_body(xt_ref, q_ref, idx_ref):
    n = xt_ref.shape[2]
    gs = idx_ref.shape[2]
    xt = xt_ref[0]
    q = q_ref[0]
    r0 = xt[0:1, :]
    r1 = xt[1:2, :]
    r2 = xt[2:3, :]
    q0 = q[:, 0:1]
    q1 = q[:, 1:2]
    q2 = q[:, 2:3]
    xtb = xt.astype(jnp.bfloat16).astype(jnp.float32)
    qb = q.astype(jnp.bfloat16).astype(jnp.float32)
    b0 = xtb[0:1, :]
    b1 = xtb[1:2, :]
    b2 = xtb[2:3, :]
    p0 = qb[:, 0:1]
    p1 = qb[:, 1:2]
    p2 = qb[:, 2:3]
    qr = p0 * b0 + p1 * b1 + p2 * b2
    qq = q0 * q0 + q1 * q1 + q2 * q2
    rr = r0 * r0 + r1 * r1 + r2 * r2
    d = (qq - 2.0 * qr) + rr
    lane = lax.broadcasted_iota(jnp.int32, d.shape, 1)
    for j in range(gs):
        m = jnp.min(d, axis=1, keepdims=True)
        sel = jnp.min(jnp.where(d == m, lane, n), axis=1, keepdims=True)
        idx_ref[0, :, j:j + 1] = sel
        d = jnp.where(lane == sel, jnp.inf, d)


def _make_sc_gather(ncloud, n, nidx):
    mesh = plsc.VectorSubcoreMesh(core_axis_name="c", subcore_axis_name="s")

    @functools.partial(
        pl.kernel,
        out_type=jax.ShapeDtypeStruct((ncloud, 3 * nidx), jnp.float32),
        mesh=mesh,
        scratch_types=[
            pltpu.VMEM((3 * n,), jnp.float32),
            pltpu.VMEM((nidx,), jnp.int32),
            pltpu.VMEM((3 * nidx,), jnp.float32),
            pltpu.VMEM((3 * nidx,), jnp.float32),
        ],
        compiler_params=pltpu.CompilerParams(needs_layout_passes=False),
    )
    def sc_gather(x_hbm, idx_hbm, cexp_hbm, out_hbm, xv, iv, cv, ov):
        w = lax.axis_index("s") * 2 + lax.axis_index("c")
        pltpu.sync_copy(x_hbm.at[w], xv)
        pltpu.sync_copy(idx_hbm.at[w], iv)
        pltpu.sync_copy(cexp_hbm.at[w], cv)

        def step(i, carry):
            off = pl.multiple_of(i * 16, 16)
            ivec = iv[pl.ds(off, 16)]
            for c3 in range(3):
                g = plsc.load_gather(xv, [ivec + jnp.int32(c3 * n)])
                off2 = pl.multiple_of(c3 * nidx + i * 16, 16)
                ov[pl.ds(off2, 16)] = g - cv[pl.ds(off2, 16)]
            return carry

        lax.fori_loop(0, nidx // 16, step, 0)
        pltpu.sync_copy(ov, out_hbm.at[w])

    return sc_gather


def _build_queries(centers, t):
    step_f = int((_EXPAND - 1.0) * _NUM_GROUP / t * 2)
    step_b = int((_EXPAND - 1.0) * _NUM_GROUP)
    parts = []
    for i in range(t):
        a = centers[:, i * step_f:i * step_f + (_NUM_GROUP - step_b)]
        b = centers[:, (i - 1) * step_b + _NUM_GROUP + (t - 1) * step_f:
                    i * step_b + _NUM_GROUP + (t - 1) * step_f]
        parts.append(jnp.concatenate((a, b), axis=1))
    return jnp.stack(parts, axis=0)


def kernel(xyz):
    t, b, n, _ = xyz.shape
    gs = _GROUP_SIZE
    step_f = int((_EXPAND - 1.0) * _NUM_GROUP / t * 2)
    step_b = int((_EXPAND - 1.0) * _NUM_GROUP)
    ncen = _NUM_GROUP + (step_f + step_b) * (t - 1)
    nc = t * b

    x = xyz.reshape(nc, n, 3)
    xt = jnp.transpose(x, (0, 2, 1))

    x8t = jnp.transpose(x[:b], (2, 0, 1))
    cent = pl.pallas_call(
        _fps_body,
        out_shape=jax.ShapeDtypeStruct((3, b, ncen), jnp.float32),
    )(x8t)
    centers = jnp.transpose(cent, (1, 2, 0))

    queries = _build_queries(centers, t).reshape(nc, -1, 3)
    m = queries.shape[1]

    idx = pl.pallas_call(
        _knn_body,
        grid=(nc,),
        in_specs=[
            pl.BlockSpec((1, 3, n), lambda g: (g, 0, 0)),
            pl.BlockSpec((1, m, 3), lambda g: (g, 0, 0)),
        ],
        out_specs=pl.BlockSpec((1, m, gs), lambda g: (g, 0, 0)),
        out_shape=jax.ShapeDtypeStruct((nc, m, gs), jnp.int32),
    )(xt, queries)

    xflat = xt.reshape(nc, 3 * n)
    idxflat = idx.reshape(nc, m * gs)
    qt = jnp.transpose(queries, (0, 2, 1))
    cexp = jnp.broadcast_to(
        qt[:, :, :, None], (nc, 3, m, gs)).reshape(nc, 3 * m * gs)
    nb = _make_sc_gather(nc, n, m * gs)(xflat, idxflat, cexp)

    neighborhood = nb.reshape(nc, 3, m, gs).transpose(0, 2, 3, 1)
    neighborhood = neighborhood.reshape(t, b, m, gs, 3)
    center_out = queries.reshape(t, b, m, 3)
    return (neighborhood, center_out)
